# Initial kernel scaffold; baseline (speedup 1.0000x reference)
import jax
import jax.numpy as jnp
from jax.experimental import pallas as pl


def kernel(x, edge_index, W1, a1_src, a1_dst, W2, a2_src, a2_dst):
    raise NotImplementedError("write your pallas kernel here")


# speedup vs baseline: 26.7358x; 26.7358x over previous
import functools

import jax
import jax.numpy as jnp
from jax import lax
from jax.experimental import pallas as pl
from jax.experimental.pallas import tpu as pltpu
from jax.experimental.pallas import tpu_sc as plsc

N = 10000
E = 320000
D_IN = 128
D_HID = 64
D_OUT = 64
NEG_SLOPE = 0.2

NC = 2
NS = 16
NW = NC * NS
L = 16

CH = 128
NCHUNK = E // CH
ROWS_PT = 624
ROWS_TAIL = N - ROWS_PT * NS
N_PAD = 10240


def _tc_embed_body(x_ref, w_ref, asv_ref, adv_ref, h_ref, s_ref, d_ref):
    h = jnp.dot(x_ref[...], w_ref[...], preferred_element_type=jnp.float32)
    h_ref[...] = h
    s_ref[...] = jnp.dot(h, asv_ref[...], preferred_element_type=jnp.float32)
    d_ref[...] = jnp.dot(h, adv_ref[...], preferred_element_type=jnp.float32)


def _tc_embed(x, W, a_src, a_dst):
    n = x.shape[0]
    dh = W.shape[1]
    h, s, d = pl.pallas_call(
        _tc_embed_body,
        out_shape=[
            jax.ShapeDtypeStruct((n, dh), jnp.float32),
            jax.ShapeDtypeStruct((n, 1), jnp.float32),
            jax.ShapeDtypeStruct((n, 1), jnp.float32),
        ],
    )(x, W, a_src[:, None], a_dst[:, None])
    return h, s[:, 0], d[:, 0]


def _tc_combine_embed_body(p_ref, d_ref, w_ref, asv_ref, adv_ref,
                           h_ref, s_ref, dd_ref):
    denom = d_ref[0] + d_ref[1] + 1e-16
    z = (p_ref[0] + p_ref[1]) / denom
    g = jnp.where(z > 0, z, jnp.exp(z) - 1.0)
    h = jnp.dot(g, w_ref[...], preferred_element_type=jnp.float32)
    h_ref[...] = h
    s_ref[...] = jnp.dot(h, asv_ref[...], preferred_element_type=jnp.float32)
    dd_ref[...] = jnp.dot(h, adv_ref[...], preferred_element_type=jnp.float32)


def _tc_combine_embed(p, d, W, a_src, a_dst):
    n = p.shape[1]
    dh = W.shape[1]
    h, s, dd = pl.pallas_call(
        _tc_combine_embed_body,
        out_shape=[
            jax.ShapeDtypeStruct((n, dh), jnp.float32),
            jax.ShapeDtypeStruct((n, 1), jnp.float32),
            jax.ShapeDtypeStruct((n, 1), jnp.float32),
        ],
    )(p, d[:, :, None], W, a_src[:, None], a_dst[:, None])
    return h, s[:, 0], dd[:, 0]


def _tc_finalize_body(p_ref, d_ref, o_ref):
    denom = d_ref[0] + d_ref[1] + 1e-16
    o_ref[...] = (p_ref[0] + p_ref[1]) / denom


def _tc_finalize(p, d):
    n = p.shape[1]
    dh = p.shape[2]
    return pl.pallas_call(
        _tc_finalize_body,
        out_shape=jax.ShapeDtypeStruct((n, dh), jnp.float32),
    )(p, d[:, :, None])


def _sc_edge_body(h_hbm, src_hbm, dst_hbm, as_hbm, ad_hbm,
                  p_hbm, dn_hbm,
                  h_sp, acc_sp, den_sp,
                  as_v, ad_v, src_v, dst_v, w_v, rows_v, zd_v,
                  sem):
    cid = lax.axis_index("c")
    sid = lax.axis_index("s")
    wid = sid * NC + cid

    r0 = sid * ROWS_PT
    nfull = ROWS_PT // CH
    rem = ROWS_PT - nfull * CH

    pltpu.sync_copy(as_hbm, as_v)
    pltpu.sync_copy(ad_hbm, ad_v)

    def _stage(i, _):
        pltpu.sync_copy(h_hbm.at[pl.ds(r0 + i * CH, CH)], rows_v)
        pltpu.sync_copy(rows_v, h_sp.at[pl.ds(r0 + i * CH, CH)])
        return _
    lax.fori_loop(0, nfull, _stage, None)
    pltpu.sync_copy(h_hbm.at[pl.ds(r0 + nfull * CH, rem)],
                    rows_v.at[pl.ds(0, rem)])
    pltpu.sync_copy(rows_v.at[pl.ds(0, rem)],
                    h_sp.at[pl.ds(r0 + nfull * CH, rem)])

    @pl.when(sid == NS - 1)
    def _tail_stage():
        t0 = N - ROWS_TAIL
        pltpu.sync_copy(h_hbm.at[pl.ds(t0, ROWS_TAIL)],
                        rows_v.at[pl.ds(0, ROWS_TAIL)])
        pltpu.sync_copy(rows_v.at[pl.ds(0, ROWS_TAIL)],
                        h_sp.at[pl.ds(t0, ROWS_TAIL)])

    zeros16 = jnp.zeros((L,), jnp.float32)

    def _zrow(i, _):
        for j in range(D_HID // L):
            rows_v[i, pl.ds(j * L, L)] = zeros16
        return _
    lax.fori_loop(0, CH, _zrow, None)

    def _zd(i, _):
        zd_v[pl.ds(i * L, L)] = zeros16
        return _
    lax.fori_loop(0, ROWS_PT // L, _zd, None)

    def _zacc(i, _):
        pltpu.sync_copy(rows_v, acc_sp.at[pl.ds(r0 + i * CH, CH)])
        return _
    lax.fori_loop(0, nfull, _zacc, None)
    pltpu.sync_copy(rows_v.at[pl.ds(0, rem)],
                    acc_sp.at[pl.ds(r0 + nfull * CH, rem)])
    pltpu.sync_copy(zd_v, den_sp.at[pl.ds(r0, ROWS_PT)])

    @pl.when(sid == NS - 1)
    def _tail_zero():
        t0 = N - ROWS_TAIL
        pltpu.sync_copy(rows_v.at[pl.ds(0, ROWS_TAIL)],
                        acc_sp.at[pl.ds(t0, ROWS_TAIL)])
        pltpu.sync_copy(zd_v.at[pl.ds(0, ROWS_TAIL)],
                        den_sp.at[pl.ds(t0, ROWS_TAIL)])

    plsc.subcore_barrier()

    lo = (wid * NCHUNK) // NW
    hi = ((wid + 1) * NCHUNK) // NW

    def _chunk(ci, _):
        base = ci * CH
        pltpu.sync_copy(src_hbm.at[pl.ds(base, CH)], src_v)
        pltpu.sync_copy(dst_hbm.at[pl.ds(base, CH)], dst_v)

        gat = pltpu.async_copy(h_sp.at[src_v], rows_v, sem)

        def _wchunk(i, _):
            s16 = src_v[pl.ds(i * L, L)]
            d16 = dst_v[pl.ds(i * L, L)]
            e = plsc.load_gather(as_v, [s16]) + plsc.load_gather(ad_v, [d16])
            e = jnp.where(e > 0, e, NEG_SLOPE * e)
            w_v[pl.ds(i * L, L)] = jnp.exp(e)
            return _
        lax.fori_loop(0, CH // L, _wchunk, None)

        gat.wait()

        def _scale(i, _):
            wv = plsc.load_gather(w_v, [jnp.full((L,), i, jnp.int32)])
            for j in range(D_HID // L):
                sl = pl.ds(j * L, L)
                rows_v[i, sl] = rows_v[i, sl] * wv
            return _
        lax.fori_loop(0, CH, _scale, None)

        pltpu.sync_copy(rows_v, acc_sp.at[dst_v], add=True)
        pltpu.sync_copy(w_v, den_sp.at[dst_v], add=True)
        return _

    lax.fori_loop(lo, hi, _chunk, None)

    plsc.subcore_barrier()

    def _wb(i, _):
        pltpu.sync_copy(acc_sp.at[pl.ds(r0 + i * CH, CH)], rows_v)
        pltpu.sync_copy(rows_v, p_hbm.at[cid, pl.ds(r0 + i * CH, CH)])
        return _
    lax.fori_loop(0, nfull, _wb, None)
    pltpu.sync_copy(acc_sp.at[pl.ds(r0 + nfull * CH, rem)],
                    rows_v.at[pl.ds(0, rem)])
    pltpu.sync_copy(rows_v.at[pl.ds(0, rem)],
                    p_hbm.at[cid, pl.ds(r0 + nfull * CH, rem)])
    pltpu.sync_copy(den_sp.at[pl.ds(r0, ROWS_PT)], zd_v)
    pltpu.sync_copy(zd_v, dn_hbm.at[pl.ds(cid * N_PAD + r0, ROWS_PT)])

    @pl.when(sid == NS - 1)
    def _tail_out():
        t0 = N - ROWS_TAIL
        pltpu.sync_copy(acc_sp.at[pl.ds(t0, ROWS_TAIL)],
                        rows_v.at[pl.ds(0, ROWS_TAIL)])
        pltpu.sync_copy(rows_v.at[pl.ds(0, ROWS_TAIL)],
                        p_hbm.at[cid, pl.ds(t0, ROWS_TAIL)])
        pltpu.sync_copy(den_sp.at[pl.ds(t0, ROWS_TAIL)],
                        zd_v.at[pl.ds(0, ROWS_TAIL)])
        pltpu.sync_copy(zd_v.at[pl.ds(0, ROWS_TAIL)],
                        dn_hbm.at[pl.ds(cid * N_PAD + t0, ROWS_TAIL)])


_sc_edge = pl.kernel(
    _sc_edge_body,
    out_type=(
        jax.ShapeDtypeStruct((NC, N, D_HID), jnp.float32),
        jax.ShapeDtypeStruct((NC * N_PAD,), jnp.float32),
    ),
    mesh=plsc.VectorSubcoreMesh(
        core_axis_name="c", subcore_axis_name="s",
        num_cores=NC, num_subcores=NS),
    compiler_params=pltpu.CompilerParams(
        needs_layout_passes=False, use_tc_tiling_on_sc=False),
    scratch_types=[
        pltpu.VMEM_SHARED((N, D_HID), jnp.float32),
        pltpu.VMEM_SHARED((N, D_HID), jnp.float32),
        pltpu.VMEM_SHARED((N,), jnp.float32),
        pltpu.VMEM((N,), jnp.float32),
        pltpu.VMEM((N,), jnp.float32),
        pltpu.VMEM((CH,), jnp.int32),
        pltpu.VMEM((CH,), jnp.int32),
        pltpu.VMEM((CH,), jnp.float32),
        pltpu.VMEM((CH, D_HID), jnp.float32),
        pltpu.VMEM((ROWS_PT,), jnp.float32),
        pltpu.SemaphoreType.DMA,
    ],
)


def kernel(x, edge_index, W1, a1_src, a1_dst, W2, a2_src, a2_dst):
    src = edge_index[0]
    dst = edge_index[1]
    h1, as1, ad1 = _tc_embed(x, W1, a1_src, a1_dst)
    p1, d1f = _sc_edge(h1, src, dst, as1, ad1)
    d1 = d1f.reshape(NC, N_PAD)[:, :N]
    h2, as2, ad2 = _tc_combine_embed(p1, d1, W2, a2_src, a2_dst)
    p2, d2f = _sc_edge(h2, src, dst, as2, ad2)
    d2 = d2f.reshape(NC, N_PAD)[:, :N]
    return _tc_finalize(p2, d2)


# speedup vs baseline: 31.4674x; 1.1770x over previous
import functools

import jax
import jax.numpy as jnp
from jax import lax
from jax.experimental import pallas as pl
from jax.experimental.pallas import tpu as pltpu
from jax.experimental.pallas import tpu_sc as plsc

N = 10000
E = 320000
D_IN = 128
D_HID = 64
D_OUT = 64
NEG_SLOPE = 0.2

NC = 2
NS = 16
NW = NC * NS
L = 16

CH = 128
NCHUNK = E // CH
ROWS_PT = 624
ROWS_TAIL = N - ROWS_PT * NS
N_PAD = 10240


def _tc_embed_body(x_ref, w_ref, asv_ref, adv_ref, h_ref, s_ref, d_ref):
    h = jnp.dot(x_ref[...], w_ref[...], preferred_element_type=jnp.float32)
    h_ref[...] = h
    s_ref[...] = jnp.dot(h, asv_ref[...], preferred_element_type=jnp.float32)
    d_ref[...] = jnp.dot(h, adv_ref[...], preferred_element_type=jnp.float32)


def _tc_embed(x, W, a_src, a_dst):
    n = x.shape[0]
    dh = W.shape[1]
    h, s, d = pl.pallas_call(
        _tc_embed_body,
        out_shape=[
            jax.ShapeDtypeStruct((n, dh), jnp.float32),
            jax.ShapeDtypeStruct((n, 1), jnp.float32),
            jax.ShapeDtypeStruct((n, 1), jnp.float32),
        ],
    )(x, W, a_src[:, None], a_dst[:, None])
    return h, s[:, 0], d[:, 0]


def _tc_combine_embed_body(p_ref, d_ref, w_ref, asv_ref, adv_ref,
                           h_ref, s_ref, dd_ref):
    denom = d_ref[0] + d_ref[1] + 1e-16
    z = (p_ref[0] + p_ref[1]) / denom
    g = jnp.where(z > 0, z, jnp.exp(z) - 1.0)
    h = jnp.dot(g, w_ref[...], preferred_element_type=jnp.float32)
    h_ref[...] = h
    s_ref[...] = jnp.dot(h, asv_ref[...], preferred_element_type=jnp.float32)
    dd_ref[...] = jnp.dot(h, adv_ref[...], preferred_element_type=jnp.float32)


def _tc_combine_embed(p, d, W, a_src, a_dst):
    n = p.shape[1]
    dh = W.shape[1]
    h, s, dd = pl.pallas_call(
        _tc_combine_embed_body,
        out_shape=[
            jax.ShapeDtypeStruct((n, dh), jnp.float32),
            jax.ShapeDtypeStruct((n, 1), jnp.float32),
            jax.ShapeDtypeStruct((n, 1), jnp.float32),
        ],
    )(p, d[:, :, None], W, a_src[:, None], a_dst[:, None])
    return h, s[:, 0], dd[:, 0]


def _tc_finalize_body(p_ref, d_ref, o_ref):
    denom = d_ref[0] + d_ref[1] + 1e-16
    o_ref[...] = (p_ref[0] + p_ref[1]) / denom


def _tc_finalize(p, d):
    n = p.shape[1]
    dh = p.shape[2]
    return pl.pallas_call(
        _tc_finalize_body,
        out_shape=jax.ShapeDtypeStruct((n, dh), jnp.float32),
    )(p, d[:, :, None])


def _sc_edge_body(h_hbm, src_hbm, dst_hbm, as_hbm, ad_hbm,
                  p_hbm, dn_hbm,
                  h_sp, acc_sp, den_sp,
                  as_v, ad_v, src_v, dst_v, w_v, rows_v, zd_v,
                  sem):
    cid = lax.axis_index("c")
    sid = lax.axis_index("s")
    wid = sid * NC + cid

    r0 = sid * ROWS_PT
    nfull = ROWS_PT // CH
    rem = ROWS_PT - nfull * CH

    pltpu.sync_copy(as_hbm, as_v)
    pltpu.sync_copy(ad_hbm, ad_v)

    def _stage(i, _):
        pltpu.sync_copy(h_hbm.at[pl.ds(r0 + i * CH, CH)], rows_v)
        pltpu.sync_copy(rows_v, h_sp.at[pl.ds(r0 + i * CH, CH)])
        return _
    lax.fori_loop(0, nfull, _stage, None)
    pltpu.sync_copy(h_hbm.at[pl.ds(r0 + nfull * CH, rem)],
                    rows_v.at[pl.ds(0, rem)])
    pltpu.sync_copy(rows_v.at[pl.ds(0, rem)],
                    h_sp.at[pl.ds(r0 + nfull * CH, rem)])

    @pl.when(sid == NS - 1)
    def _tail_stage():
        t0 = N - ROWS_TAIL
        pltpu.sync_copy(h_hbm.at[pl.ds(t0, ROWS_TAIL)],
                        rows_v.at[pl.ds(0, ROWS_TAIL)])
        pltpu.sync_copy(rows_v.at[pl.ds(0, ROWS_TAIL)],
                        h_sp.at[pl.ds(t0, ROWS_TAIL)])

    zeros16 = jnp.zeros((L,), jnp.float32)

    def _zrow(i, _):
        for j in range(D_HID // L):
            rows_v[i, pl.ds(j * L, L)] = zeros16
        return _
    lax.fori_loop(0, CH, _zrow, None)

    def _zd(i, _):
        zd_v[pl.ds(i * L, L)] = zeros16
        return _
    lax.fori_loop(0, ROWS_PT // L, _zd, None)

    def _zacc(i, _):
        pltpu.sync_copy(rows_v, acc_sp.at[pl.ds(r0 + i * CH, CH)])
        return _
    lax.fori_loop(0, nfull, _zacc, None)
    pltpu.sync_copy(rows_v.at[pl.ds(0, rem)],
                    acc_sp.at[pl.ds(r0 + nfull * CH, rem)])
    pltpu.sync_copy(zd_v, den_sp.at[pl.ds(r0, ROWS_PT)])

    @pl.when(sid == NS - 1)
    def _tail_zero():
        t0 = N - ROWS_TAIL
        pltpu.sync_copy(rows_v.at[pl.ds(0, ROWS_TAIL)],
                        acc_sp.at[pl.ds(t0, ROWS_TAIL)])
        pltpu.sync_copy(zd_v.at[pl.ds(0, ROWS_TAIL)],
                        den_sp.at[pl.ds(t0, ROWS_TAIL)])

    plsc.subcore_barrier()

    lo = (wid * NCHUNK) // NW
    hi = ((wid + 1) * NCHUNK) // NW

    def _chunk(ci, _):
        base = ci * CH
        pltpu.sync_copy(src_hbm.at[pl.ds(base, CH)], src_v)
        pltpu.sync_copy(dst_hbm.at[pl.ds(base, CH)], dst_v)

        gat = pltpu.async_copy(h_sp.at[src_v], rows_v, sem)

        @plsc.parallel_loop(0, CH // L, unroll=2)
        def _wchunk(i):
            s16 = src_v[pl.ds(i * L, L)]
            d16 = dst_v[pl.ds(i * L, L)]
            e = plsc.load_gather(as_v, [s16]) + plsc.load_gather(ad_v, [d16])
            e = jnp.where(e > 0, e, NEG_SLOPE * e)
            w_v[pl.ds(i * L, L)] = jnp.exp(e)

        gat.wait()

        @plsc.parallel_loop(0, CH, unroll=4)
        def _scale(i):
            wv = plsc.load_gather(w_v, [jnp.full((L,), i, jnp.int32)])
            for j in range(D_HID // L):
                sl = pl.ds(j * L, L)
                rows_v[i, sl] = rows_v[i, sl] * wv

        pltpu.sync_copy(rows_v, acc_sp.at[dst_v], add=True)
        pltpu.sync_copy(w_v, den_sp.at[dst_v], add=True)
        return _

    lax.fori_loop(lo, hi, _chunk, None)

    plsc.subcore_barrier()

    def _wb(i, _):
        pltpu.sync_copy(acc_sp.at[pl.ds(r0 + i * CH, CH)], rows_v)
        pltpu.sync_copy(rows_v, p_hbm.at[cid, pl.ds(r0 + i * CH, CH)])
        return _
    lax.fori_loop(0, nfull, _wb, None)
    pltpu.sync_copy(acc_sp.at[pl.ds(r0 + nfull * CH, rem)],
                    rows_v.at[pl.ds(0, rem)])
    pltpu.sync_copy(rows_v.at[pl.ds(0, rem)],
                    p_hbm.at[cid, pl.ds(r0 + nfull * CH, rem)])
    pltpu.sync_copy(den_sp.at[pl.ds(r0, ROWS_PT)], zd_v)
    pltpu.sync_copy(zd_v, dn_hbm.at[pl.ds(cid * N_PAD + r0, ROWS_PT)])

    @pl.when(sid == NS - 1)
    def _tail_out():
        t0 = N - ROWS_TAIL
        pltpu.sync_copy(acc_sp.at[pl.ds(t0, ROWS_TAIL)],
                        rows_v.at[pl.ds(0, ROWS_TAIL)])
        pltpu.sync_copy(rows_v.at[pl.ds(0, ROWS_TAIL)],
                        p_hbm.at[cid, pl.ds(t0, ROWS_TAIL)])
        pltpu.sync_copy(den_sp.at[pl.ds(t0, ROWS_TAIL)],
                        zd_v.at[pl.ds(0, ROWS_TAIL)])
        pltpu.sync_copy(zd_v.at[pl.ds(0, ROWS_TAIL)],
                        dn_hbm.at[pl.ds(cid * N_PAD + t0, ROWS_TAIL)])


_sc_edge = pl.kernel(
    _sc_edge_body,
    out_type=(
        jax.ShapeDtypeStruct((NC, N, D_HID), jnp.float32),
        jax.ShapeDtypeStruct((NC * N_PAD,), jnp.float32),
    ),
    mesh=plsc.VectorSubcoreMesh(
        core_axis_name="c", subcore_axis_name="s",
        num_cores=NC, num_subcores=NS),
    compiler_params=pltpu.CompilerParams(
        needs_layout_passes=False, use_tc_tiling_on_sc=False),
    scratch_types=[
        pltpu.VMEM_SHARED((N, D_HID), jnp.float32),
        pltpu.VMEM_SHARED((N, D_HID), jnp.float32),
        pltpu.VMEM_SHARED((N,), jnp.float32),
        pltpu.VMEM((N,), jnp.float32),
        pltpu.VMEM((N,), jnp.float32),
        pltpu.VMEM((CH,), jnp.int32),
        pltpu.VMEM((CH,), jnp.int32),
        pltpu.VMEM((CH,), jnp.float32),
        pltpu.VMEM((CH, D_HID), jnp.float32),
        pltpu.VMEM((ROWS_PT,), jnp.float32),
        pltpu.SemaphoreType.DMA,
    ],
)


def kernel(x, edge_index, W1, a1_src, a1_dst, W2, a2_src, a2_dst):
    src = edge_index[0]
    dst = edge_index[1]
    h1, as1, ad1 = _tc_embed(x, W1, a1_src, a1_dst)
    p1, d1f = _sc_edge(h1, src, dst, as1, ad1)
    d1 = d1f.reshape(NC, N_PAD)[:, :N]
    h2, as2, ad2 = _tc_combine_embed(p1, d1, W2, a2_src, a2_dst)
    p2, d2f = _sc_edge(h2, src, dst, as2, ad2)
    d2 = d2f.reshape(NC, N_PAD)[:, :N]
    return _tc_finalize(p2, d2)


# speedup vs baseline: 46.9781x; 1.4929x over previous
import functools

import jax
import jax.numpy as jnp
from jax import lax
from jax.experimental import pallas as pl
from jax.experimental.pallas import tpu as pltpu
from jax.experimental.pallas import tpu_sc as plsc

N = 10000
E = 320000
D_IN = 128
D_HID = 64
D_OUT = 64
NEG_SLOPE = 0.2

NC = 2
NS = 16
NW = NC * NS
L = 16

CH = 128
EPT = E // NW
NFULL = EPT // CH
REM = EPT - NFULL * CH
NB = 3
SB = 48
ZD = 208
ROWS_PT = 624
ROWS_TAIL = N - ROWS_PT * NS
N_PAD = 10240


def _tc_embed_body(x_ref, w_ref, asv_ref, adv_ref, h_ref, s_ref, d_ref):
    h = jnp.dot(x_ref[...], w_ref[...], preferred_element_type=jnp.float32)
    h_ref[...] = h
    s_ref[...] = jnp.dot(h, asv_ref[...], preferred_element_type=jnp.float32)
    d_ref[...] = jnp.dot(h, adv_ref[...], preferred_element_type=jnp.float32)


def _tc_embed(x, W, a_src, a_dst):
    n = x.shape[0]
    dh = W.shape[1]
    h, s, d = pl.pallas_call(
        _tc_embed_body,
        out_shape=[
            jax.ShapeDtypeStruct((n, dh), jnp.float32),
            jax.ShapeDtypeStruct((n, 1), jnp.float32),
            jax.ShapeDtypeStruct((n, 1), jnp.float32),
        ],
    )(x, W, a_src[:, None], a_dst[:, None])
    return h, s[:, 0], d[:, 0]


def _tc_combine_embed_body(p_ref, d_ref, w_ref, asv_ref, adv_ref,
                           h_ref, s_ref, dd_ref):
    denom = d_ref[0] + d_ref[1] + 1e-16
    z = (p_ref[0] + p_ref[1]) / denom
    g = jnp.where(z > 0, z, jnp.exp(z) - 1.0)
    h = jnp.dot(g, w_ref[...], preferred_element_type=jnp.float32)
    h_ref[...] = h
    s_ref[...] = jnp.dot(h, asv_ref[...], preferred_element_type=jnp.float32)
    dd_ref[...] = jnp.dot(h, adv_ref[...], preferred_element_type=jnp.float32)


def _tc_combine_embed(p, d, W, a_src, a_dst):
    n = p.shape[1]
    dh = W.shape[1]
    h, s, dd = pl.pallas_call(
        _tc_combine_embed_body,
        out_shape=[
            jax.ShapeDtypeStruct((n, dh), jnp.float32),
            jax.ShapeDtypeStruct((n, 1), jnp.float32),
            jax.ShapeDtypeStruct((n, 1), jnp.float32),
        ],
    )(p, d[:, :, None], W, a_src[:, None], a_dst[:, None])
    return h, s[:, 0], dd[:, 0]


def _tc_finalize_body(p_ref, d_ref, o_ref):
    denom = d_ref[0] + d_ref[1] + 1e-16
    o_ref[...] = (p_ref[0] + p_ref[1]) / denom


def _tc_finalize(p, d):
    n = p.shape[1]
    dh = p.shape[2]
    return pl.pallas_call(
        _tc_finalize_body,
        out_shape=jax.ShapeDtypeStruct((n, dh), jnp.float32),
    )(p, d[:, :, None])


def _sc_edge_body(h_hbm, src_hbm, dst_hbm, as_hbm, ad_hbm,
                  p_hbm, dn_hbm,
                  h_sp, acc_sp, den_sp,
                  as_v, ad_v, srcq, dstq, w_q, rowsq, rows_v,
                  rsrc, rdst, rw, rrows, zd_v,
                  isem, gsem, ssem):
    cid = lax.axis_index("c")
    sid = lax.axis_index("s")
    wid = sid * NC + cid

    r0 = sid * ROWS_PT
    nsb = ROWS_PT // SB

    pltpu.sync_copy(as_hbm, as_v)
    pltpu.sync_copy(ad_hbm, ad_v)

    def _stage(i, _):
        pltpu.sync_copy(h_hbm.at[pl.ds(r0 + i * SB, SB)], rows_v)
        pltpu.sync_copy(rows_v, h_sp.at[pl.ds(r0 + i * SB, SB)])
        return _
    lax.fori_loop(0, nsb, _stage, None)

    @pl.when(sid == NS - 1)
    def _tail_stage():
        t0 = N - ROWS_TAIL
        pltpu.sync_copy(h_hbm.at[pl.ds(t0, ROWS_TAIL)],
                        rows_v.at[pl.ds(0, ROWS_TAIL)])
        pltpu.sync_copy(rows_v.at[pl.ds(0, ROWS_TAIL)],
                        h_sp.at[pl.ds(t0, ROWS_TAIL)])

    zeros16 = jnp.zeros((L,), jnp.float32)

    def _zrow(i, _):
        for j in range(D_HID // L):
            rows_v[i, pl.ds(j * L, L)] = zeros16
        return _
    lax.fori_loop(0, SB, _zrow, None)

    def _zd(i, _):
        zd_v[pl.ds(i * L, L)] = zeros16
        return _
    lax.fori_loop(0, ZD // L, _zd, None)

    def _zacc(i, _):
        pltpu.sync_copy(rows_v, acc_sp.at[pl.ds(r0 + i * SB, SB)])
        return _
    lax.fori_loop(0, nsb, _zacc, None)

    def _zden(i, _):
        pltpu.sync_copy(zd_v, den_sp.at[pl.ds(r0 + i * ZD, ZD)])
        return _
    lax.fori_loop(0, ROWS_PT // ZD, _zden, None)

    @pl.when(sid == NS - 1)
    def _tail_zero():
        t0 = N - ROWS_TAIL
        pltpu.sync_copy(rows_v.at[pl.ds(0, ROWS_TAIL)],
                        acc_sp.at[pl.ds(t0, ROWS_TAIL)])
        pltpu.sync_copy(zd_v.at[pl.ds(0, ROWS_TAIL)],
                        den_sp.at[pl.ds(t0, ROWS_TAIL)])

    plsc.subcore_barrier()

    e0 = wid * EPT

    def _issue_idx(c, b):
        base = e0 + c * CH
        pltpu.async_copy(src_hbm.at[pl.ds(base, CH)], srcq.at[b], isem.at[b])
        pltpu.async_copy(dst_hbm.at[pl.ds(base, CH)], dstq.at[b], isem.at[b])

    def _wait_idx(c, b):
        base = e0 + c * CH
        pltpu.make_async_copy(
            src_hbm.at[pl.ds(base, CH)], srcq.at[b], isem.at[b]).wait()
        pltpu.make_async_copy(
            dst_hbm.at[pl.ds(base, CH)], dstq.at[b], isem.at[b]).wait()

    def _issue_gather(b):
        pltpu.async_copy(h_sp.at[srcq.at[b]], rowsq.at[b], gsem.at[b])

    def _wait_gather(b):
        pltpu.make_async_copy(
            h_sp.at[srcq.at[b]], rowsq.at[b], gsem.at[b]).wait()

    def _issue_scatter(b):
        pltpu.async_copy(rowsq.at[b], acc_sp.at[dstq.at[b]], ssem.at[b],
                         add=True)
        pltpu.async_copy(w_q.at[b], den_sp.at[dstq.at[b]], ssem.at[b],
                         add=True)

    def _drain_scatter(b):
        pltpu.make_async_copy(
            rowsq.at[b], acc_sp.at[dstq.at[b]], ssem.at[b]).wait()
        pltpu.make_async_copy(
            w_q.at[b], den_sp.at[dstq.at[b]], ssem.at[b]).wait()

    def _compute_w(b):
        @plsc.parallel_loop(0, CH // L, unroll=2)
        def _w(i):
            s16 = srcq[b, pl.ds(i * L, L)]
            d16 = dstq[b, pl.ds(i * L, L)]
            e = plsc.load_gather(as_v, [s16]) + plsc.load_gather(ad_v, [d16])
            e = jnp.where(e > 0, e, NEG_SLOPE * e)
            w_q[b, pl.ds(i * L, L)] = jnp.exp(e)

    def _scale_rows(b):
        @plsc.parallel_loop(0, CH, unroll=4)
        def _s(i):
            wv = plsc.load_gather(w_q.at[b], [jnp.full((L,), i, jnp.int32)])
            for j in range(D_HID // L):
                sl = pl.ds(j * L, L)
                rowsq[b, i, sl] = rowsq[b, i, sl] * wv

    _issue_idx(0, 0)
    _issue_idx(1, 1)
    _wait_idx(0, 0)
    _issue_gather(0)

    def _chunk(ci, _):
        b = lax.rem(ci, NB)
        s1 = lax.rem(ci + 1, NB)
        s2 = lax.rem(ci + 2, NB)
        _compute_w(b)
        _wait_gather(b)
        _scale_rows(b)
        _issue_scatter(b)

        @pl.when(ci + 1 < NFULL)
        def _prep_gather():
            _wait_idx(ci + 1, s1)
            _issue_gather(s1)

        @pl.when(ci >= NB - 2)
        def _drain_prev():
            _drain_scatter(s2)

        @pl.when(ci + 2 < NFULL)
        def _prep_idx():
            _issue_idx(ci + 2, s2)
        return _

    lax.fori_loop(0, NFULL, _chunk, None)

    for cd in range(NFULL - (NB - 2), NFULL):
        _drain_scatter(jnp.int32(cd % NB))

    rbase = e0 + NFULL * CH
    pltpu.sync_copy(src_hbm.at[pl.ds(rbase, REM)], rsrc)
    pltpu.sync_copy(dst_hbm.at[pl.ds(rbase, REM)], rdst)
    s16 = rsrc[...]
    d16 = rdst[...]
    e = plsc.load_gather(as_v, [s16]) + plsc.load_gather(ad_v, [d16])
    e = jnp.where(e > 0, e, NEG_SLOPE * e)
    rw[...] = jnp.exp(e)
    pltpu.sync_copy(h_sp.at[rsrc], rrows)

    @plsc.parallel_loop(0, REM, unroll=4)
    def _rscale(i):
        wv = plsc.load_gather(rw, [jnp.full((L,), i, jnp.int32)])
        for j in range(D_HID // L):
            sl = pl.ds(j * L, L)
            rrows[i, sl] = rrows[i, sl] * wv

    pltpu.sync_copy(rrows, acc_sp.at[rdst], add=True)
    pltpu.sync_copy(rw, den_sp.at[rdst], add=True)

    plsc.subcore_barrier()

    def _wb(i, _):
        pltpu.sync_copy(acc_sp.at[pl.ds(r0 + i * SB, SB)], rows_v)
        pltpu.sync_copy(rows_v, p_hbm.at[cid, pl.ds(r0 + i * SB, SB)])
        return _
    lax.fori_loop(0, nsb, _wb, None)

    def _wbden(i, _):
        pltpu.sync_copy(den_sp.at[pl.ds(r0 + i * ZD, ZD)], zd_v)
        pltpu.sync_copy(zd_v, dn_hbm.at[pl.ds(cid * N_PAD + r0 + i * ZD, ZD)])
        return _
    lax.fori_loop(0, ROWS_PT // ZD, _wbden, None)

    @pl.when(sid == NS - 1)
    def _tail_out():
        t0 = N - ROWS_TAIL
        pltpu.sync_copy(acc_sp.at[pl.ds(t0, ROWS_TAIL)],
                        rows_v.at[pl.ds(0, ROWS_TAIL)])
        pltpu.sync_copy(rows_v.at[pl.ds(0, ROWS_TAIL)],
                        p_hbm.at[cid, pl.ds(t0, ROWS_TAIL)])
        pltpu.sync_copy(den_sp.at[pl.ds(t0, ROWS_TAIL)],
                        zd_v.at[pl.ds(0, ROWS_TAIL)])
        pltpu.sync_copy(zd_v.at[pl.ds(0, ROWS_TAIL)],
                        dn_hbm.at[pl.ds(cid * N_PAD + t0, ROWS_TAIL)])


_sc_edge = pl.kernel(
    _sc_edge_body,
    out_type=(
        jax.ShapeDtypeStruct((NC, N, D_HID), jnp.float32),
        jax.ShapeDtypeStruct((NC * N_PAD,), jnp.float32),
    ),
    mesh=plsc.VectorSubcoreMesh(
        core_axis_name="c", subcore_axis_name="s",
        num_cores=NC, num_subcores=NS),
    compiler_params=pltpu.CompilerParams(
        needs_layout_passes=False, use_tc_tiling_on_sc=False),
    scratch_types=[
        pltpu.VMEM_SHARED((N, D_HID), jnp.float32),
        pltpu.VMEM_SHARED((N, D_HID), jnp.float32),
        pltpu.VMEM_SHARED((N,), jnp.float32),
        pltpu.VMEM((N,), jnp.float32),
        pltpu.VMEM((N,), jnp.float32),
        pltpu.VMEM((NB, CH), jnp.int32),
        pltpu.VMEM((NB, CH), jnp.int32),
        pltpu.VMEM((NB, CH), jnp.float32),
        pltpu.VMEM((NB, CH, D_HID), jnp.float32),
        pltpu.VMEM((SB, D_HID), jnp.float32),
        pltpu.VMEM((REM,), jnp.int32),
        pltpu.VMEM((REM,), jnp.int32),
        pltpu.VMEM((REM,), jnp.float32),
        pltpu.VMEM((REM, D_HID), jnp.float32),
        pltpu.VMEM((ZD,), jnp.float32),
        pltpu.SemaphoreType.DMA((NB,)),
        pltpu.SemaphoreType.DMA((NB,)),
        pltpu.SemaphoreType.DMA((NB,)),
    ],
)


def kernel(x, edge_index, W1, a1_src, a1_dst, W2, a2_src, a2_dst):
    src = edge_index[0]
    dst = edge_index[1]
    h1, as1, ad1 = _tc_embed(x, W1, a1_src, a1_dst)
    p1, d1f = _sc_edge(h1, src, dst, as1, ad1)
    d1 = d1f.reshape(NC, N_PAD)[:, :N]
    h2, as2, ad2 = _tc_combine_embed(p1, d1, W2, a2_src, a2_dst)
    p2, d2f = _sc_edge(h2, src, dst, as2, ad2)
    d2 = d2f.reshape(NC, N_PAD)[:, :N]
    return _tc_finalize(p2, d2)
